# baseline (speedup 1.0000x reference)
import functools

import jax
import jax.numpy as jnp
from jax import lax
from jax.experimental import pallas as pl
from jax.experimental.pallas import tpu as pltpu
from jax.experimental.pallas import tpu_sc as plsc

N = 10000
E = 320000
F = 128
H = 128
GED = 64

NC, NS, L = 2, 16, 16
NW = NC * NS
EPT = E // NW
CB = 128
NCH = EPT // CB
TAIL = EPT - NCH * CB
NIB = 4
STEP = 624
SW = 640

_mesh = plsc.VectorSubcoreMesh(core_axis_name="c", subcore_axis_name="s",
                               num_cores=NC, num_subcores=NS)
_sc_params = pltpu.CompilerParams(needs_layout_passes=False)


@functools.partial(
    pl.kernel,
    out_type=jax.ShapeDtypeStruct((NW, 1, N), jnp.float32),
    mesh=_mesh,
    compiler_params=_sc_params,
    scratch_types=[
        pltpu.VMEM((EPT,), jnp.int32),
        pltpu.VMEM((EPT,), jnp.float32),
        pltpu.VMEM((N,), jnp.float32),
    ],
)
def _deg_kernel(dst_hbm, ew_hbm, deg_out, dst_v, ew_v, acc_v):
    c = lax.axis_index("c")
    s = lax.axis_index("s")
    w = c * NS + s
    zero = jnp.zeros((L,), jnp.float32)

    def zloop(i, carry):
        acc_v[pl.ds(i * L, L)] = zero
        return carry

    lax.fori_loop(0, N // L, zloop, 0)
    base = w * EPT
    pltpu.sync_copy(dst_hbm.at[pl.ds(base, EPT)], dst_v)
    pltpu.sync_copy(ew_hbm.at[pl.ds(base, EPT)], ew_v)

    def grp(g, carry):
        dv = dst_v[pl.ds(g * L, L)]
        ev = ew_v[pl.ds(g * L, L)]
        plsc.addupdate_scatter(acc_v, [dv], ev)
        return carry

    lax.fori_loop(0, EPT // L, grp, 0)
    pltpu.sync_copy(acc_v, deg_out.at[w, 0])


@functools.partial(
    pl.kernel,
    out_type=jax.ShapeDtypeStruct((NC, N, H), jnp.float32),
    mesh=_mesh,
    compiler_params=_sc_params,
    scratch_types=[
        [pltpu.VMEM((CB,), jnp.int32) for _ in range(NIB)],
        [pltpu.VMEM((CB,), jnp.int32) for _ in range(NIB)],
        [pltpu.VMEM((CB,), jnp.float32) for _ in range(NIB)],
        pltpu.VMEM((N,), jnp.float32),
        pltpu.VMEM((2 * CB, H), jnp.float32),
        pltpu.VMEM((TAIL,), jnp.int32),
        pltpu.VMEM((TAIL,), jnp.int32),
        pltpu.VMEM((TAIL,), jnp.float32),
        pltpu.VMEM_SHARED((N, H), jnp.float32),
        pltpu.SemaphoreType.DMA((2,)),
        pltpu.SemaphoreType.DMA((2,)),
        pltpu.SemaphoreType.DMA((NIB,)),
    ],
)
def _conv_kernel(h_hbm, src_hbm, dst_hbm, ew_hbm, dis_hbm, zeros_hbm, part_out,
                 src_c, dst_c, ew_c, dis_v, ring, src_t, dst_t, ew_t, acc,
                 sem_g, sem_s, sem_i):
    c = lax.axis_index("c")
    s = lax.axis_index("s")
    w = c * NS + s
    base = w * EPT
    rbase = s * STEP
    d1 = pltpu.async_copy(dis_hbm, dis_v, sem_g.at[0])
    d5 = pltpu.async_copy(zeros_hbm, acc.at[pl.ds(rbase, SW)], sem_g.at[1])

    def issue_idx(ch, j):
        off = base + ch * CB
        pltpu.async_copy(src_hbm.at[pl.ds(off, CB)], src_c[j], sem_i.at[j])
        pltpu.async_copy(dst_hbm.at[pl.ds(off, CB)], dst_c[j], sem_i.at[j])
        pltpu.async_copy(ew_hbm.at[pl.ds(off, CB)], ew_c[j], sem_i.at[j])

    def wait_idx(j):
        pltpu.make_async_copy(src_hbm.at[pl.ds(0, CB)], src_c[j],
                              sem_i.at[j]).wait()
        pltpu.make_async_copy(src_hbm.at[pl.ds(0, CB)], dst_c[j],
                              sem_i.at[j]).wait()
        pltpu.make_async_copy(ew_hbm.at[pl.ds(0, CB)], ew_c[j],
                              sem_i.at[j]).wait()

    issue_idx(0, 0)
    issue_idx(1, 1)
    d1.wait()
    d5.wait()
    plsc.subcore_barrier()
    wait_idx(0)
    pltpu.async_copy(h_hbm.at[src_c[0]], ring.at[pl.ds(0, CB)], sem_g.at[0])
    zidx = jnp.zeros((L,), jnp.int32)

    def _process_chunk(ch, b, j, guard_first, issue_next, last):
        rsl = pl.ds(b * CB, CB)
        nrsl = pl.ds((1 - b) * CB, CB)

        def _free_other_slot():
            pltpu.make_async_copy(ring.at[nrsl], acc.at[dst_c[0]],
                                  sem_s.at[1 - b]).wait()

        if not last:
            wait_idx((j + 1) % NIB)
            if guard_first:
                pl.when(ch >= 1)(_free_other_slot)
            else:
                _free_other_slot()
            pltpu.async_copy(h_hbm.at[src_c[(j + 1) % NIB]], ring.at[nrsl],
                             sem_g.at[1 - b])
            if issue_next:
                issue_idx(ch + 2, (j + 2) % NIB)
        else:
            _free_other_slot()
        for g in range(CB // L):
            sl = pl.ds(g * L, L)
            sv = src_c[j][sl]
            dv = dst_c[j][sl]
            ev = ew_c[j][sl]
            ew_c[j][sl] = (plsc.load_gather(dis_v, [sv]) * ev
                           * plsc.load_gather(dis_v, [dv]))
        pltpu.make_async_copy(h_hbm.at[src_c[0]], ring.at[rsl],
                              sem_g.at[b]).wait()
        zsp = jnp.zeros((L,), jnp.int32)

        def edge(i, icarry):
            for q in range(4):
                e = i * 4 + q
                nb = plsc.load_gather(ew_c[j], [zsp + e])
                ro = b * CB + e
                for k in range(H // L):
                    sl = (ro, pl.ds(k * L, L))
                    ring[sl] = ring[sl] * nb
            return icarry

        lax.fori_loop(0, CB // 4, edge, 0)
        pltpu.async_copy(ring.at[rsl], acc.at[dst_c[j]], sem_s.at[b],
                         add=True)

    def chunk4(i, carry):
        for p in range(4):
            _process_chunk(i * 4 + p, p & 1, p, p == 0, True, False)
        return carry

    _NPIPE = ((NCH - 2) // 4) * 4
    lax.fori_loop(0, _NPIPE // 4, chunk4, 0)
    for ch in range(_NPIPE, NCH):
        _process_chunk(ch, ch & 1, ch % NIB, False, ch + 2 < NCH,
                       ch == NCH - 1)
    pltpu.make_async_copy(ring.at[pl.ds(((NCH - 1) & 1) * CB, CB)],
                          acc.at[dst_c[0]], sem_s.at[(NCH - 1) & 1]).wait()

    toff = base + NCH * CB
    pltpu.sync_copy(src_hbm.at[pl.ds(toff, TAIL)], src_t)
    pltpu.sync_copy(dst_hbm.at[pl.ds(toff, TAIL)], dst_t)
    pltpu.sync_copy(ew_hbm.at[pl.ds(toff, TAIL)], ew_t)
    sv = src_t[...]
    dv = dst_t[...]
    ev = ew_t[...]
    nv = plsc.load_gather(dis_v, [sv]) * ev * plsc.load_gather(dis_v, [dv])
    ew_t[...] = nv
    pltpu.async_copy(h_hbm.at[sv], ring.at[pl.ds(0, L)], sem_g.at[0])
    pltpu.make_async_copy(h_hbm.at[zidx], ring.at[pl.ds(0, L)],
                          sem_g.at[0]).wait()
    zsp = jnp.zeros((L,), jnp.int32)

    def tedge(e, carry):
        nb = plsc.load_gather(ew_t, [zsp + e])
        for k in range(H // L):
            sl = (e, pl.ds(k * L, L))
            ring[sl] = ring[sl] * nb
        return carry

    lax.fori_loop(0, TAIL, tedge, 0)
    pltpu.sync_copy(ring.at[pl.ds(0, L)], acc.at[dv], add=True)

    plsc.subcore_barrier()
    pltpu.sync_copy(acc.at[pl.ds(rbase, SW)],
                    part_out.at[c, pl.ds(rbase, SW)])


def _dis_body(dp_ref, o_ref):
    deg = jnp.sum(dp_ref[...], axis=1, keepdims=True) + 1.0
    o_ref[...] = lax.rsqrt(deg)


_dis_kernel = pl.pallas_call(
    _dis_body,
    grid=(1,),
    in_specs=[pl.BlockSpec((N, NW), lambda i: (0, 0))],
    out_specs=pl.BlockSpec((N, 1), lambda i: (0, 0)),
    out_shape=jax.ShapeDtypeStruct((N, 1), jnp.float32),
)

_BM = 2000


def _mm_body(x_ref, w_ref, o_ref):
    o_ref[...] = jnp.dot(x_ref[...], w_ref[...],
                         preferred_element_type=jnp.float32)


_mm_kernel = pl.pallas_call(
    _mm_body,
    grid=(N // _BM,),
    in_specs=[pl.BlockSpec((_BM, F), lambda i: (i, 0)),
              pl.BlockSpec((F, H), lambda i: (0, 0))],
    out_specs=pl.BlockSpec((_BM, H), lambda i: (i, 0)),
    out_shape=jax.ShapeDtypeStruct((N, H), jnp.float32),
)


def _comb1_body(p_ref, h_ref, dis_ref, b1_ref, g_ref, be_ref, w2_ref, o_ref):
    d = dis_ref[...]
    t = p_ref[0] + p_ref[1] + h_ref[...] * (d * d) + b1_ref[...]
    t = jnp.maximum(t, 0.0)
    mu = jnp.mean(t, axis=1, keepdims=True)
    dev = t - mu
    var = jnp.mean(dev * dev, axis=1, keepdims=True)
    t = dev * lax.rsqrt(var + 1e-5) * g_ref[...] + be_ref[...]
    o_ref[...] = jnp.dot(t, w2_ref[...], preferred_element_type=jnp.float32)


_comb1_kernel = pl.pallas_call(
    _comb1_body,
    grid=(N // _BM,),
    in_specs=[pl.BlockSpec((NC, _BM, H), lambda i: (0, i, 0)),
              pl.BlockSpec((_BM, H), lambda i: (i, 0)),
              pl.BlockSpec((_BM, 1), lambda i: (i, 0)),
              pl.BlockSpec((1, H), lambda i: (0, 0)),
              pl.BlockSpec((1, H), lambda i: (0, 0)),
              pl.BlockSpec((1, H), lambda i: (0, 0)),
              pl.BlockSpec((H, H), lambda i: (0, 0))],
    out_specs=pl.BlockSpec((_BM, H), lambda i: (i, 0)),
    out_shape=jax.ShapeDtypeStruct((N, H), jnp.float32),
)


def _comb2_body(p_ref, h_ref, dis_ref, b2_ref, o_ref):
    d = dis_ref[...]
    o_ref[...] = p_ref[0] + p_ref[1] + h_ref[...] * (d * d) + b2_ref[...]


_comb2_kernel = pl.pallas_call(
    _comb2_body,
    grid=(N // _BM,),
    in_specs=[pl.BlockSpec((NC, _BM, H), lambda i: (0, i, 0)),
              pl.BlockSpec((_BM, H), lambda i: (i, 0)),
              pl.BlockSpec((_BM, 1), lambda i: (i, 0)),
              pl.BlockSpec((1, H), lambda i: (0, 0))],
    out_specs=pl.BlockSpec((_BM, H), lambda i: (i, 0)),
    out_shape=jax.ShapeDtypeStruct((N, H), jnp.float32),
)

_AM = 400


def _adj_body(zi_ref, zt_ref, o_ref):
    o_ref[...] = jnp.dot(zi_ref[...], zt_ref[...],
                         preferred_element_type=jnp.float32)


_adj_kernel = pl.pallas_call(
    _adj_body,
    grid=(N // _AM,),
    in_specs=[pl.BlockSpec((_AM, GED), lambda i: (i, 0)),
              pl.BlockSpec((GED, N), lambda i: (0, 0))],
    out_specs=pl.BlockSpec((_AM, N), lambda i: (i, 0)),
    out_shape=jax.ShapeDtypeStruct((N, N), jnp.float32),
)


def kernel(x, edge_index, edge_weight, W1, b1, gamma, beta, W2, b2):
    src = edge_index[0]
    dst = edge_index[1]
    ew = edge_weight

    deg_part = _deg_kernel(dst, ew)
    dis = _dis_kernel(deg_part.reshape(NW, N).T)
    dis1d = dis.reshape(N)
    h1 = _mm_kernel(x, W1)
    zeros = jnp.zeros((SW, H), jnp.float32)

    p1 = _conv_kernel(h1, src, dst, ew, dis1d, zeros)
    h2m = _comb1_kernel(p1, h1, dis, b1.reshape(1, H), gamma.reshape(1, H),
                        beta.reshape(1, H), W2)
    p2 = _conv_kernel(h2m, src, dst, ew, dis1d, zeros)
    out2 = _comb2_kernel(p2, h2m, dis, b2.reshape(1, H))

    mu = out2[:, :GED]
    log_var = out2[:, GED:]
    z = mu
    adj = _adj_kernel(z, z.T)
    return adj, mu, log_var, z


# speedup vs baseline: 1.1611x; 1.1611x over previous
import functools

import jax
import jax.numpy as jnp
from jax import lax
from jax.experimental import pallas as pl
from jax.experimental.pallas import tpu as pltpu
from jax.experimental.pallas import tpu_sc as plsc

N = 10000
E = 320000
F = 128
H = 128
GED = 64

NC, NS, L = 2, 16, 16
NW = NC * NS
EPT = E // NW
NGRP = EPT // L
G = 8
D = 6
STEP = 624
SW = 640

_mesh = plsc.VectorSubcoreMesh(core_axis_name="c", subcore_axis_name="s",
                               num_cores=NC, num_subcores=NS)
_sc_params = pltpu.CompilerParams(needs_layout_passes=False)


@functools.partial(
    pl.kernel,
    out_type=jax.ShapeDtypeStruct((NW, 1, N), jnp.float32),
    mesh=_mesh,
    compiler_params=_sc_params,
    scratch_types=[
        pltpu.VMEM((EPT,), jnp.int32),
        pltpu.VMEM((EPT,), jnp.float32),
        pltpu.VMEM((N,), jnp.float32),
    ],
)
def _deg_kernel(dst_hbm, ew_hbm, deg_out, dst_v, ew_v, acc_v):
    c = lax.axis_index("c")
    s = lax.axis_index("s")
    w = c * NS + s
    zero = jnp.zeros((L,), jnp.float32)

    def zloop(i, carry):
        acc_v[pl.ds(i * L, L)] = zero
        return carry

    lax.fori_loop(0, N // L, zloop, 0)
    base = w * EPT
    pltpu.sync_copy(dst_hbm.at[pl.ds(base, EPT)], dst_v)
    pltpu.sync_copy(ew_hbm.at[pl.ds(base, EPT)], ew_v)

    def grp(g, carry):
        dv = dst_v[pl.ds(g * L, L)]
        ev = ew_v[pl.ds(g * L, L)]
        plsc.addupdate_scatter(acc_v, [dv], ev)
        return carry

    lax.fori_loop(0, EPT // L, grp, 0)
    pltpu.sync_copy(acc_v, deg_out.at[w, 0])


@functools.partial(
    pl.kernel,
    out_type=jax.ShapeDtypeStruct((NC, N, H), jnp.float32),
    mesh=_mesh,
    compiler_params=_sc_params,
    scratch_types=[
        pltpu.VMEM((EPT,), jnp.int32),
        pltpu.VMEM((EPT,), jnp.int32),
        pltpu.VMEM((EPT,), jnp.float32),
        pltpu.VMEM((G * L, H), jnp.float32),
        pltpu.VMEM_SHARED((N, H), jnp.float32),
        pltpu.SemaphoreType.DMA((G,)),
        pltpu.SemaphoreType.DMA((G,)),
    ],
)
def _conv_kernel(h_hbm, src_hbm, dst_hbm, ew_hbm, dis_hbm, zeros_hbm, part_out,
                 src_v, dst_v, norm_v, ring, acc, sem_g, sem_s):
    c = lax.axis_index("c")
    s = lax.axis_index("s")
    w = c * NS + s
    base = w * EPT
    rbase = s * STEP
    d1 = pltpu.async_copy(dis_hbm, ring.at[pl.ds(0, 80)], sem_g.at[0])
    d2 = pltpu.async_copy(src_hbm.at[pl.ds(base, EPT)], src_v, sem_g.at[1])
    d3 = pltpu.async_copy(dst_hbm.at[pl.ds(base, EPT)], dst_v, sem_g.at[2])
    d4 = pltpu.async_copy(ew_hbm.at[pl.ds(base, EPT)], norm_v, sem_g.at[3])
    d5 = pltpu.async_copy(zeros_hbm, acc.at[pl.ds(rbase, SW)], sem_g.at[4])
    d1.wait()
    d2.wait()
    d3.wait()
    d4.wait()
    d5.wait()

    def norm_loop(g, carry):
        sl = pl.ds(g * L, L)
        sv = src_v[sl]
        dv = dst_v[sl]
        ev = norm_v[sl]
        ds_ = plsc.load_gather(ring, [lax.shift_right_logical(sv, 7),
                                      lax.bitwise_and(sv, 127)])
        dd = plsc.load_gather(ring, [lax.shift_right_logical(dv, 7),
                                     lax.bitwise_and(dv, 127)])
        norm_v[sl] = ds_ * ev * dd
        return carry

    lax.fori_loop(0, NGRP, norm_loop, 0)
    plsc.subcore_barrier()

    for g in range(D):
        sv = src_v[pl.ds(g * L, L)]
        pltpu.async_copy(h_hbm.at[sv], ring.at[pl.ds(g * L, L)],
                         sem_g.at[g])

    zidx = jnp.zeros((L,), jnp.int32)

    def _process_group(g, slot):
        rsl = pl.ds(slot * L, L)
        pltpu.make_async_copy(h_hbm.at[zidx], ring.at[rsl],
                              sem_g.at[slot]).wait()
        ebase = g * L
        eidx = jnp.full((L,), ebase, jnp.int32)

        def edge(i, icarry):
            for q in range(4):
                e = i * 4 + q
                nb = plsc.load_gather(norm_v, [eidx + e])
                ro = slot * L + e
                for k in range(H // L):
                    sl = (ro, pl.ds(k * L, L))
                    ring[sl] = ring[sl] * nb
            return icarry

        lax.fori_loop(0, L // 4, edge, 0)
        dv = dst_v[pl.ds(ebase, L)]
        pltpu.async_copy(ring.at[rsl], acc.at[dv], sem_s.at[slot],
                         add=True)

        nslot = (slot + D) % G
        nsl = pl.ds(nslot * L, L)

        @pl.when(g + D < NGRP)
        def _():
            @pl.when(g >= G - D)
            def _():
                pltpu.make_async_copy(ring.at[nsl], acc.at[zidx],
                                      sem_s.at[nslot]).wait()

            nsv = src_v[pl.ds((g + D) * L, L)]
            pltpu.async_copy(h_hbm.at[nsv], ring.at[nsl], sem_g.at[nslot])

    def grp8(i, carry):
        for q in range(G):
            _process_group(i * G + q, q)
        return carry

    lax.fori_loop(0, NGRP // G, grp8, 0)
    for q in range(NGRP % G):
        _process_group((NGRP // G) * G + q, q)
    for slot in range(G):
        off = slot * L
        pltpu.make_async_copy(ring.at[pl.ds(off, L)], acc.at[zidx],
                              sem_s.at[slot]).wait()
    plsc.subcore_barrier()
    pltpu.sync_copy(acc.at[pl.ds(rbase, SW)],
                    part_out.at[c, pl.ds(rbase, SW)])


def _dis_body(dp_ref, o_ref):
    deg = jnp.sum(dp_ref[...], axis=1, keepdims=True) + 1.0
    o_ref[...] = lax.rsqrt(deg)


_dis_kernel = pl.pallas_call(
    _dis_body,
    grid=(1,),
    in_specs=[pl.BlockSpec((N, NW), lambda i: (0, 0))],
    out_specs=pl.BlockSpec((N, 1), lambda i: (0, 0)),
    out_shape=jax.ShapeDtypeStruct((N, 1), jnp.float32),
)

_BM = 2000


def _mm_body(x_ref, w_ref, o_ref):
    o_ref[...] = jnp.dot(x_ref[...], w_ref[...],
                         preferred_element_type=jnp.float32)


_mm_kernel = pl.pallas_call(
    _mm_body,
    grid=(N // _BM,),
    in_specs=[pl.BlockSpec((_BM, F), lambda i: (i, 0)),
              pl.BlockSpec((F, H), lambda i: (0, 0))],
    out_specs=pl.BlockSpec((_BM, H), lambda i: (i, 0)),
    out_shape=jax.ShapeDtypeStruct((N, H), jnp.float32),
)


def _comb1_body(p_ref, h_ref, dis_ref, b1_ref, g_ref, be_ref, w2_ref, o_ref):
    d = dis_ref[...]
    t = p_ref[0] + p_ref[1] + h_ref[...] * (d * d) + b1_ref[...]
    t = jnp.maximum(t, 0.0)
    mu = jnp.mean(t, axis=1, keepdims=True)
    dev = t - mu
    var = jnp.mean(dev * dev, axis=1, keepdims=True)
    t = dev * lax.rsqrt(var + 1e-5) * g_ref[...] + be_ref[...]
    o_ref[...] = jnp.dot(t, w2_ref[...], preferred_element_type=jnp.float32)


_comb1_kernel = pl.pallas_call(
    _comb1_body,
    grid=(N // _BM,),
    in_specs=[pl.BlockSpec((NC, _BM, H), lambda i: (0, i, 0)),
              pl.BlockSpec((_BM, H), lambda i: (i, 0)),
              pl.BlockSpec((_BM, 1), lambda i: (i, 0)),
              pl.BlockSpec((1, H), lambda i: (0, 0)),
              pl.BlockSpec((1, H), lambda i: (0, 0)),
              pl.BlockSpec((1, H), lambda i: (0, 0)),
              pl.BlockSpec((H, H), lambda i: (0, 0))],
    out_specs=pl.BlockSpec((_BM, H), lambda i: (i, 0)),
    out_shape=jax.ShapeDtypeStruct((N, H), jnp.float32),
)


def _comb2_body(p_ref, h_ref, dis_ref, b2_ref, o_ref):
    d = dis_ref[...]
    o_ref[...] = p_ref[0] + p_ref[1] + h_ref[...] * (d * d) + b2_ref[...]


_comb2_kernel = pl.pallas_call(
    _comb2_body,
    grid=(N // _BM,),
    in_specs=[pl.BlockSpec((NC, _BM, H), lambda i: (0, i, 0)),
              pl.BlockSpec((_BM, H), lambda i: (i, 0)),
              pl.BlockSpec((_BM, 1), lambda i: (i, 0)),
              pl.BlockSpec((1, H), lambda i: (0, 0))],
    out_specs=pl.BlockSpec((_BM, H), lambda i: (i, 0)),
    out_shape=jax.ShapeDtypeStruct((N, H), jnp.float32),
)

_AM = 400


def _adj_body(zi_ref, zt_ref, o_ref):
    o_ref[...] = jnp.dot(zi_ref[...], zt_ref[...],
                         preferred_element_type=jnp.float32)


_adj_kernel = pl.pallas_call(
    _adj_body,
    grid=(N // _AM,),
    in_specs=[pl.BlockSpec((_AM, GED), lambda i: (i, 0)),
              pl.BlockSpec((GED, N), lambda i: (0, 0))],
    out_specs=pl.BlockSpec((_AM, N), lambda i: (i, 0)),
    out_shape=jax.ShapeDtypeStruct((N, N), jnp.float32),
)


def kernel(x, edge_index, edge_weight, W1, b1, gamma, beta, W2, b2):
    src = edge_index[0]
    dst = edge_index[1]
    ew = edge_weight

    deg_part = _deg_kernel(dst, ew)
    dis = _dis_kernel(deg_part.reshape(NW, N).T)
    dis_pad = jnp.concatenate(
        [dis.reshape(N), jnp.zeros((80 * H - N,), jnp.float32)]
    ).reshape(80, H)
    h1 = _mm_kernel(x, W1)
    zeros = jnp.zeros((SW, H), jnp.float32)

    p1 = _conv_kernel(h1, src, dst, ew, dis_pad, zeros)
    h2m = _comb1_kernel(p1, h1, dis, b1.reshape(1, H), gamma.reshape(1, H),
                        beta.reshape(1, H), W2)
    p2 = _conv_kernel(h2m, src, dst, ew, dis_pad, zeros)
    out2 = _comb2_kernel(p2, h2m, dis, b2.reshape(1, H))

    mu = out2[:, :GED]
    log_var = out2[:, GED:]
    z = mu
    adj = _adj_kernel(z, z.T)
    return adj, mu, log_var, z


# speedup vs baseline: 1.1855x; 1.0210x over previous
import functools

import jax
import jax.numpy as jnp
from jax import lax
from jax.experimental import pallas as pl
from jax.experimental.pallas import tpu as pltpu
from jax.experimental.pallas import tpu_sc as plsc

N = 10000
E = 320000
F = 128
H = 128
GED = 64

NC, NS, L = 2, 16, 16
NW = NC * NS
EPT = E // NW
NGRP = EPT // L
G = 8
D = 7
STEP = 624
SW = 640

_mesh = plsc.VectorSubcoreMesh(core_axis_name="c", subcore_axis_name="s",
                               num_cores=NC, num_subcores=NS)
_sc_params = pltpu.CompilerParams(needs_layout_passes=False)


@functools.partial(
    pl.kernel,
    out_type=jax.ShapeDtypeStruct((NW, 1, N), jnp.float32),
    mesh=_mesh,
    compiler_params=_sc_params,
    scratch_types=[
        pltpu.VMEM((EPT,), jnp.int32),
        pltpu.VMEM((EPT,), jnp.float32),
        pltpu.VMEM((N,), jnp.float32),
    ],
)
def _deg_kernel(dst_hbm, ew_hbm, deg_out, dst_v, ew_v, acc_v):
    c = lax.axis_index("c")
    s = lax.axis_index("s")
    w = c * NS + s
    zero = jnp.zeros((L,), jnp.float32)

    def zloop(i, carry):
        acc_v[pl.ds(i * L, L)] = zero
        return carry

    lax.fori_loop(0, N // L, zloop, 0)
    base = w * EPT
    pltpu.sync_copy(dst_hbm.at[pl.ds(base, EPT)], dst_v)
    pltpu.sync_copy(ew_hbm.at[pl.ds(base, EPT)], ew_v)

    def grp(g, carry):
        dv = dst_v[pl.ds(g * L, L)]
        ev = ew_v[pl.ds(g * L, L)]
        plsc.addupdate_scatter(acc_v, [dv], ev)
        return carry

    lax.fori_loop(0, EPT // L, grp, 0)
    pltpu.sync_copy(acc_v, deg_out.at[w, 0])


@functools.partial(
    pl.kernel,
    out_type=jax.ShapeDtypeStruct((NC, N, H), jnp.float32),
    mesh=_mesh,
    compiler_params=_sc_params,
    scratch_types=[
        pltpu.VMEM((EPT,), jnp.int32),
        pltpu.VMEM((EPT,), jnp.int32),
        pltpu.VMEM((EPT,), jnp.float32),
        pltpu.VMEM((G * L, H), jnp.float32),
        pltpu.VMEM_SHARED((N, H), jnp.float32),
        pltpu.SemaphoreType.DMA((G,)),
        pltpu.SemaphoreType.DMA((G,)),
    ],
)
def _conv_kernel(h_hbm, src_hbm, dst_hbm, ew_hbm, dis_hbm, zeros_hbm, part_out,
                 src_v, dst_v, norm_v, ring, acc, sem_g, sem_s):
    c = lax.axis_index("c")
    s = lax.axis_index("s")
    w = c * NS + s
    base = w * EPT
    rbase = s * STEP
    d1 = pltpu.async_copy(dis_hbm, ring.at[pl.ds(0, 80)], sem_g.at[0])
    d2 = pltpu.async_copy(src_hbm.at[pl.ds(base, EPT)], src_v, sem_g.at[1])
    d3 = pltpu.async_copy(dst_hbm.at[pl.ds(base, EPT)], dst_v, sem_g.at[2])
    d4 = pltpu.async_copy(ew_hbm.at[pl.ds(base, EPT)], norm_v, sem_g.at[3])
    d5 = pltpu.async_copy(zeros_hbm, acc.at[pl.ds(rbase, SW)], sem_g.at[4])
    d1.wait()
    d2.wait()
    d3.wait()
    d4.wait()
    d5.wait()

    def norm_loop(g, carry):
        sl = pl.ds(g * L, L)
        sv = src_v[sl]
        dv = dst_v[sl]
        ev = norm_v[sl]
        ds_ = plsc.load_gather(ring, [lax.shift_right_logical(sv, 7),
                                      lax.bitwise_and(sv, 127)])
        dd = plsc.load_gather(ring, [lax.shift_right_logical(dv, 7),
                                     lax.bitwise_and(dv, 127)])
        norm_v[sl] = ds_ * ev * dd
        return carry

    lax.fori_loop(0, NGRP, norm_loop, 0)
    plsc.subcore_barrier()

    for g in range(D):
        sv = src_v[pl.ds(g * L, L)]
        pltpu.async_copy(h_hbm.at[sv], ring.at[pl.ds(g * L, L)],
                         sem_g.at[g])

    zidx = jnp.zeros((L,), jnp.int32)

    def _process_group(g, slot):
        rsl = pl.ds(slot * L, L)
        pltpu.make_async_copy(h_hbm.at[zidx], ring.at[rsl],
                              sem_g.at[slot]).wait()
        ebase = g * L
        eidx = jnp.full((L,), ebase, jnp.int32)

        def edge(i, icarry):
            for q in range(4):
                e = i * 4 + q
                nb = plsc.load_gather(norm_v, [eidx + e])
                ro = slot * L + e
                for k in range(H // L):
                    sl = (ro, pl.ds(k * L, L))
                    ring[sl] = ring[sl] * nb
            return icarry

        lax.fori_loop(0, L // 4, edge, 0)
        dv = dst_v[pl.ds(ebase, L)]
        pltpu.async_copy(ring.at[rsl], acc.at[dv], sem_s.at[slot],
                         add=True)

        nslot = (slot + D) % G
        nsl = pl.ds(nslot * L, L)

        @pl.when(g + D < NGRP)
        def _():
            @pl.when(g >= G - D)
            def _():
                pltpu.make_async_copy(ring.at[nsl], acc.at[zidx],
                                      sem_s.at[nslot]).wait()

            nsv = src_v[pl.ds((g + D) * L, L)]
            pltpu.async_copy(h_hbm.at[nsv], ring.at[nsl], sem_g.at[nslot])

    def grp8(i, carry):
        for q in range(G):
            _process_group(i * G + q, q)
        return carry

    lax.fori_loop(0, NGRP // G, grp8, 0)
    for q in range(NGRP % G):
        _process_group((NGRP // G) * G + q, q)
    for slot in range(G):
        off = slot * L
        pltpu.make_async_copy(ring.at[pl.ds(off, L)], acc.at[zidx],
                              sem_s.at[slot]).wait()
    plsc.subcore_barrier()
    pltpu.sync_copy(acc.at[pl.ds(rbase, SW)],
                    part_out.at[c, pl.ds(rbase, SW)])


def _dis_body(dp_ref, o_ref):
    deg = jnp.sum(dp_ref[...], axis=1, keepdims=True) + 1.0
    o_ref[...] = lax.rsqrt(deg)


_dis_kernel = pl.pallas_call(
    _dis_body,
    grid=(1,),
    in_specs=[pl.BlockSpec((N, NW), lambda i: (0, 0))],
    out_specs=pl.BlockSpec((N, 1), lambda i: (0, 0)),
    out_shape=jax.ShapeDtypeStruct((N, 1), jnp.float32),
)

_BM = 2000


def _mm_body(x_ref, w_ref, o_ref):
    o_ref[...] = jnp.dot(x_ref[...], w_ref[...],
                         preferred_element_type=jnp.float32)


_mm_kernel = pl.pallas_call(
    _mm_body,
    grid=(N // _BM,),
    in_specs=[pl.BlockSpec((_BM, F), lambda i: (i, 0)),
              pl.BlockSpec((F, H), lambda i: (0, 0))],
    out_specs=pl.BlockSpec((_BM, H), lambda i: (i, 0)),
    out_shape=jax.ShapeDtypeStruct((N, H), jnp.float32),
)


def _comb1_body(p_ref, h_ref, dis_ref, b1_ref, g_ref, be_ref, w2_ref, o_ref):
    d = dis_ref[...]
    t = p_ref[0] + p_ref[1] + h_ref[...] * (d * d) + b1_ref[...]
    t = jnp.maximum(t, 0.0)
    mu = jnp.mean(t, axis=1, keepdims=True)
    dev = t - mu
    var = jnp.mean(dev * dev, axis=1, keepdims=True)
    t = dev * lax.rsqrt(var + 1e-5) * g_ref[...] + be_ref[...]
    o_ref[...] = jnp.dot(t, w2_ref[...], preferred_element_type=jnp.float32)


_comb1_kernel = pl.pallas_call(
    _comb1_body,
    grid=(N // _BM,),
    in_specs=[pl.BlockSpec((NC, _BM, H), lambda i: (0, i, 0)),
              pl.BlockSpec((_BM, H), lambda i: (i, 0)),
              pl.BlockSpec((_BM, 1), lambda i: (i, 0)),
              pl.BlockSpec((1, H), lambda i: (0, 0)),
              pl.BlockSpec((1, H), lambda i: (0, 0)),
              pl.BlockSpec((1, H), lambda i: (0, 0)),
              pl.BlockSpec((H, H), lambda i: (0, 0))],
    out_specs=pl.BlockSpec((_BM, H), lambda i: (i, 0)),
    out_shape=jax.ShapeDtypeStruct((N, H), jnp.float32),
)


def _comb2_body(p_ref, h_ref, dis_ref, b2_ref, o_ref):
    d = dis_ref[...]
    o_ref[...] = p_ref[0] + p_ref[1] + h_ref[...] * (d * d) + b2_ref[...]


_comb2_kernel = pl.pallas_call(
    _comb2_body,
    grid=(N // _BM,),
    in_specs=[pl.BlockSpec((NC, _BM, H), lambda i: (0, i, 0)),
              pl.BlockSpec((_BM, H), lambda i: (i, 0)),
              pl.BlockSpec((_BM, 1), lambda i: (i, 0)),
              pl.BlockSpec((1, H), lambda i: (0, 0))],
    out_specs=pl.BlockSpec((_BM, H), lambda i: (i, 0)),
    out_shape=jax.ShapeDtypeStruct((N, H), jnp.float32),
)

_AM = 400


def _adj_body(zi_ref, zt_ref, o_ref):
    o_ref[...] = jnp.dot(zi_ref[...], zt_ref[...],
                         preferred_element_type=jnp.float32)


_adj_kernel = pl.pallas_call(
    _adj_body,
    grid=(N // _AM,),
    in_specs=[pl.BlockSpec((_AM, GED), lambda i: (i, 0)),
              pl.BlockSpec((GED, N), lambda i: (0, 0))],
    out_specs=pl.BlockSpec((_AM, N), lambda i: (i, 0)),
    out_shape=jax.ShapeDtypeStruct((N, N), jnp.float32),
)


def kernel(x, edge_index, edge_weight, W1, b1, gamma, beta, W2, b2):
    src = edge_index[0]
    dst = edge_index[1]
    ew = edge_weight

    deg_part = _deg_kernel(dst, ew)
    dis = _dis_kernel(deg_part.reshape(NW, N).T)
    dis_pad = jnp.concatenate(
        [dis.reshape(N), jnp.zeros((80 * H - N,), jnp.float32)]
    ).reshape(80, H)
    h1 = _mm_kernel(x, W1)
    zeros = jnp.zeros((SW, H), jnp.float32)

    p1 = _conv_kernel(h1, src, dst, ew, dis_pad, zeros)
    h2m = _comb1_kernel(p1, h1, dis, b1.reshape(1, H), gamma.reshape(1, H),
                        beta.reshape(1, H), W2)
    p2 = _conv_kernel(h2m, src, dst, ew, dis_pad, zeros)
    out2 = _comb2_kernel(p2, h2m, dis, b2.reshape(1, H))

    mu = out2[:, :GED]
    log_var = out2[:, GED:]
    z = mu
    adj = _adj_kernel(z, z.T)
    return adj, mu, log_var, z
